# baseline (speedup 1.0000x reference)
import functools

import jax
import jax.numpy as jnp
from jax import lax
from jax.experimental import pallas as pl
from jax.experimental.pallas import tpu as pltpu
from jax.experimental.pallas import tpu_sc as plsc

N = 10000
NP = 10240
E = 320000
EROWS = 2560
EP = EROWS * 128
D = 128
H = 512
G = 64
PAD_IDX = N
BN_EPS = 1e-5

RPT = EROWS // 32
BM = 256
NBLK = NP // BM

_mesh = plsc.VectorSubcoreMesh(core_axis_name="c", subcore_axis_name="s")
f32 = jnp.float32


def _make_gather(width, bq):
    nb = EP // (32 * bq)

    @functools.partial(
        pl.kernel,
        out_type=jax.ShapeDtypeStruct((EP, width), f32),
        mesh=_mesh,
        scratch_types=[
            pltpu.VMEM((bq,), jnp.int32),
            pltpu.VMEM((bq, width), f32),
            pltpu.SemaphoreType.DMA,
        ],
    )
    def gather(table_hbm, src_hbm, msg_out, src_i, rows_v, sem):
        c = lax.axis_index("c")
        t = lax.axis_index("s")
        base = (t * 2 + c) * nb

        def body(jb, carry):
            eoff = pl.multiple_of((base + jb) * bq, 8)
            pltpu.sync_copy(src_hbm.at[pl.ds(eoff, bq)], src_i)
            pltpu.async_copy(table_hbm.at[src_i], rows_v, sem).wait()
            pltpu.sync_copy(rows_v, msg_out.at[pl.ds(eoff, bq)])
            return carry

        lax.fori_loop(0, nb, body, 0)

    return gather


_gather_d = _make_gather(D, 128)
_gather_h = _make_gather(H, 128)


def _mlp_body(x, s, inv, Wa, ba, Wb, bb, gsc, be, out):
    h0 = x[...] + s[...] * inv[:, 0:1]
    a = jnp.maximum(jnp.dot(h0, Wa[...], preferred_element_type=f32,
                    precision=lax.Precision.HIGHEST)
                    + ba[0], 0.0)
    b = jnp.maximum(jnp.dot(a, Wb[...], preferred_element_type=f32,
                    precision=lax.Precision.HIGHEST)
                    + bb[0], 0.0)
    out[...] = b * gsc[0] + be[0]


def _w_spec(shape):
    nd = len(shape)
    return pl.BlockSpec(shape, lambda i, _n=nd: (0,) * _n)


def _make_mlp(k_in):
    return pl.pallas_call(
        _mlp_body,
        grid=(NBLK,),
        in_specs=[
            pl.BlockSpec((BM, k_in), lambda i: (i, 0)),
            pl.BlockSpec((BM, k_in), lambda i: (i, 0)),
            pl.BlockSpec((BM, 16), lambda i: (i, 0)),
            _w_spec((k_in, H)),
            _w_spec((1, H)),
            _w_spec((H, H)),
            _w_spec((1, H)),
            _w_spec((1, H)),
            _w_spec((1, H)),
        ],
        out_specs=pl.BlockSpec((BM, H), lambda i: (i, 0)),
        out_shape=jax.ShapeDtypeStruct((NP, H), f32),
    )


_mlp_d = _make_mlp(D)
_mlp_h = _make_mlp(H)


def _pool_body(hc, b3, Wl1, bl1, Wl2, bl2, out, pacc):
    i = pl.program_id(0)

    @pl.when(i == 0)
    def _init():
        pacc[...] = jnp.zeros((G, H), f32)

    iota_g = lax.broadcasted_iota(jnp.int32, (G, BM), 0)
    onehot_t = (b3[0] == iota_g).astype(f32)
    pacc[...] += lax.dot_general(
        onehot_t, hc[...], (((1,), (0,)), ((), ())),
        preferred_element_type=f32,
                    precision=lax.Precision.HIGHEST)

    @pl.when(i == NBLK - 1)
    def _readout():
        p = pacc[...]
        r = jnp.maximum(jnp.dot(p, Wl1[...], preferred_element_type=f32,
                    precision=lax.Precision.HIGHEST)
                        + bl1[0], 0.0)
        out[...] = jnp.dot(r, Wl2[...], preferred_element_type=f32,
                    precision=lax.Precision.HIGHEST) + bl2[0]


_pool_call = pl.pallas_call(
    _pool_body,
    grid=(NBLK,),
    in_specs=[
        pl.BlockSpec((BM, H), lambda i: (i, 0)),
        pl.BlockSpec((1, 1, BM), lambda i: (i, 0, 0)),
        _w_spec((H, H)),
        _w_spec((1, H)),
        _w_spec((H, 1)),
        _w_spec((1, 1)),
    ],
    out_specs=pl.BlockSpec((G, 1), lambda i: (0, 0)),
    out_shape=jax.ShapeDtypeStruct((G, 1), f32),
    scratch_shapes=[pltpu.VMEM((G, H), f32)],
)


def _pad_rows(a):
    return jnp.concatenate(
        [a, jnp.zeros((NP - N,) + a.shape[1:], a.dtype)], axis=0)


def kernel(x, edge_index, batch, W1a, b1a, W1b, b1b, g1, be1,
           W2a, b2a, W2b, b2b, g2, be2,
           W3a, b3a, W3b, b3b, g3, be3,
           Wl1, bl1, Wl2, bl2):
    src = edge_index[0].astype(jnp.int32)
    dst = edge_index[1]
    src_flat = jnp.concatenate(
        [src, jnp.full((EP - E,), PAD_IDX, jnp.int32)])
    x_pad = _pad_rows(x)
    batch3 = jnp.concatenate(
        [batch.astype(jnp.int32), jnp.full((NP - N,), G, jnp.int32)]
    ).reshape(NBLK, 1, BM)

    cnt = jax.ops.segment_sum(jnp.ones((E,), f32), dst, num_segments=N)
    inv16 = _pad_rows(
        jnp.broadcast_to((1.0 / jnp.maximum(cnt, 1.0))[:, None], (N, 16)))
    bn = 1.0 / jnp.sqrt(1.0 + BN_EPS)

    def row(v):
        return v.reshape(1, -1)

    h = x_pad
    gathers = (_gather_d, _gather_h, _gather_h)
    mlps = (_mlp_d, _mlp_h, _mlp_h)
    params = ((W1a, b1a, W1b, b1b, g1, be1),
              (W2a, b2a, W2b, b2b, g2, be2),
              (W3a, b3a, W3b, b3b, g3, be3))
    for gat, mlp, (Wa, ba, Wb, bb, g, be) in zip(gathers, mlps, params):
        msg = gat(h, src_flat)
        s = jax.ops.segment_sum(msg[:E], dst, num_segments=N)
        h = mlp(x_pad if gat is _gather_d else h, _pad_rows(s), inv16,
                Wa, row(ba), Wb, row(bb), row(g * bn), row(be))

    return _pool_call(h, batch3, Wl1, row(bl1), Wl2, row(bl2))
